# baseline (speedup 1.0000x reference)
import jax
import jax.numpy as jnp
from jax.experimental import pallas as pl
from jax.experimental.pallas import tpu as pltpu

_ROWS = 16384
_COLS = 128
_HEAD = 8
_STRIPES = 4
_STRIPE_ROWS = (_ROWS - _HEAD) // _STRIPES


def _copy_set_kernel(x_hbm, o_hbm, head_vmem, head_in_sem, head_out_sem,
                     stripe_sems):
    for s in range(_STRIPES):
        base = _HEAD + s * _STRIPE_ROWS
        pltpu.make_async_copy(
            x_hbm.at[pl.ds(base, _STRIPE_ROWS), :],
            o_hbm.at[pl.ds(base, _STRIPE_ROWS), :],
            stripe_sems.at[s],
        ).start()

    head_in = pltpu.make_async_copy(
        x_hbm.at[pl.ds(0, _HEAD), :], head_vmem, head_in_sem)
    head_in.start()
    head_in.wait()
    col = jax.lax.broadcasted_iota(jnp.int32, (1, _COLS), 1)
    head_vmem[0:1, :] = jnp.where(col == 0, 3.0, head_vmem[0:1, :])
    head_out = pltpu.make_async_copy(
        head_vmem, o_hbm.at[pl.ds(0, _HEAD), :], head_out_sem)
    head_out.start()
    head_out.wait()

    for s in range(_STRIPES):
        base = _HEAD + s * _STRIPE_ROWS
        pltpu.make_async_copy(
            x_hbm.at[pl.ds(base, _STRIPE_ROWS), :],
            o_hbm.at[pl.ds(base, _STRIPE_ROWS), :],
            stripe_sems.at[s],
        ).wait()


def kernel(x):
    return pl.pallas_call(
        _copy_set_kernel,
        in_specs=[pl.BlockSpec(memory_space=pl.ANY)],
        out_specs=pl.BlockSpec(memory_space=pl.ANY),
        out_shape=jax.ShapeDtypeStruct((_ROWS, _COLS), jnp.float32),
        scratch_shapes=[
            pltpu.VMEM((_HEAD, _COLS), jnp.float32),
            pltpu.SemaphoreType.DMA,
            pltpu.SemaphoreType.DMA,
            pltpu.SemaphoreType.DMA((_STRIPES,)),
        ],
    )(x)


# speedup vs baseline: 19.9849x; 19.9849x over previous
import jax
import jax.numpy as jnp
from jax.experimental import pallas as pl

_ROWS = 16384
_COLS = 128
_BLOCK_ROWS = 1024


def _copy_set_kernel(x_ref, o_ref):
    o_ref[...] = x_ref[...]

    @pl.when(pl.program_id(0) == 0)
    def _():
        col = jax.lax.broadcasted_iota(jnp.int32, (1, _COLS), 1)
        o_ref[0:1, :] = jnp.where(col == 0, 3.0, x_ref[0:1, :])


def kernel(x):
    grid = (_ROWS // _BLOCK_ROWS,)
    return pl.pallas_call(
        _copy_set_kernel,
        grid=grid,
        in_specs=[pl.BlockSpec((_BLOCK_ROWS, _COLS), lambda i: (i, 0))],
        out_specs=pl.BlockSpec((_BLOCK_ROWS, _COLS), lambda i: (i, 0)),
        out_shape=jax.ShapeDtypeStruct((_ROWS, _COLS), jnp.float32),
    )(x)


# speedup vs baseline: 34.6147x; 1.7320x over previous
import jax
import jax.numpy as jnp
from jax.experimental import pallas as pl

_ROWS = 16384
_COLS = 128
_BLOCK_ROWS = 4096


def _copy_set_kernel(x_ref, o_ref):
    o_ref[...] = x_ref[...]

    @pl.when(pl.program_id(0) == 0)
    def _():
        col = jax.lax.broadcasted_iota(jnp.int32, (1, _COLS), 1)
        o_ref[0:1, :] = jnp.where(col == 0, 3.0, x_ref[0:1, :])


def kernel(x):
    grid = (_ROWS // _BLOCK_ROWS,)
    return pl.pallas_call(
        _copy_set_kernel,
        grid=grid,
        in_specs=[pl.BlockSpec((_BLOCK_ROWS, _COLS), lambda i: (i, 0))],
        out_specs=pl.BlockSpec((_BLOCK_ROWS, _COLS), lambda i: (i, 0)),
        out_shape=jax.ShapeDtypeStruct((_ROWS, _COLS), jnp.float32),
    )(x)


# speedup vs baseline: 42.4422x; 1.2261x over previous
import jax
import jax.numpy as jnp
from jax.experimental import pallas as pl

_ROWS = 16384
_COLS = 128
_BLOCK_ROWS = 8192


def _copy_set_kernel(x_ref, o_ref):
    o_ref[...] = x_ref[...]

    @pl.when(pl.program_id(0) == 0)
    def _():
        col = jax.lax.broadcasted_iota(jnp.int32, (1, _COLS), 1)
        o_ref[0:1, :] = jnp.where(col == 0, 3.0, x_ref[0:1, :])


def kernel(x):
    grid = (_ROWS // _BLOCK_ROWS,)
    return pl.pallas_call(
        _copy_set_kernel,
        grid=grid,
        in_specs=[pl.BlockSpec((_BLOCK_ROWS, _COLS), lambda i: (i, 0))],
        out_specs=pl.BlockSpec((_BLOCK_ROWS, _COLS), lambda i: (i, 0)),
        out_shape=jax.ShapeDtypeStruct((_ROWS, _COLS), jnp.float32),
    )(x)
